# baseline (speedup 1.0000x reference)
import functools

import jax
import jax.numpy as jnp
from jax import lax
from jax.experimental import pallas as pl
from jax.experimental.pallas import tpu as pltpu
from jax.experimental.pallas import tpu_sc as plsc

_NW = 32
_CH = 128


def _sc_gather_rows(table, idx2d):
    nch = idx2d.shape[0]
    d = table.shape[1]
    nblk = nch // 8
    per_w = -(-nblk // _NW)
    mesh = plsc.VectorSubcoreMesh(core_axis_name="c", subcore_axis_name="s")

    @functools.partial(
        pl.kernel,
        out_type=jax.ShapeDtypeStruct((nch * _CH, d), jnp.float32),
        mesh=mesh,
        scratch_types=[
            pltpu.VMEM((8, _CH), jnp.int32),
            pltpu.VMEM((8 * _CH, d), jnp.float32),
            pltpu.SemaphoreType.DMA,
        ],
        compiler_params=pltpu.CompilerParams(use_tc_tiling_on_sc=False),
    )
    def gather_kernel(table_hbm, idx_hbm, out_hbm, idx_v, rows_v, sem):
        wid = lax.axis_index("s") * 2 + lax.axis_index("c")
        b0 = wid * per_w

        def body(b, carry):
            blk = b0 + b

            @pl.when(blk < nblk)
            def _():
                pltpu.sync_copy(idx_hbm.at[pl.ds(blk * 8, 8)], idx_v)
                copies = [
                    pltpu.make_async_copy(
                        table_hbm.at[idx_v.at[j]],
                        rows_v.at[pl.ds(j * _CH, _CH)], sem)
                    for j in range(8)
                ]
                for c in copies:
                    c.start()
                for c in copies:
                    c.wait()
                pltpu.sync_copy(rows_v,
                                out_hbm.at[pl.ds(blk * 8 * _CH, 8 * _CH)])

            return carry

        lax.fori_loop(0, per_w, body, 0)

    return gather_kernel(table, idx2d)


def _mm(x, w, b=None, leaky=False, tile=512):
    mp, kdim = x.shape
    n = w.shape[1]
    grid = (mp // tile,)

    def body(*refs):
        if b is not None:
            x_ref, w_ref, b_ref, o_ref = refs
        else:
            x_ref, w_ref, o_ref = refs
        acc = jnp.dot(x_ref[...], w_ref[...],
                      preferred_element_type=jnp.float32)
        if b is not None:
            acc = acc + b_ref[...]
        if leaky:
            acc = jnp.where(acc > 0, acc, 0.1 * acc)
        o_ref[...] = acc

    in_specs = [
        pl.BlockSpec((tile, kdim), lambda i: (i, 0)),
        pl.BlockSpec((kdim, n), lambda i: (0, 0)),
    ]
    args = [x, w]
    if b is not None:
        in_specs.append(pl.BlockSpec((1, n), lambda i: (0, 0)))
        args.append(b.reshape(1, n))
    return pl.pallas_call(
        body,
        grid=grid,
        in_specs=in_specs,
        out_specs=pl.BlockSpec((tile, n), lambda i: (i, 0)),
        out_shape=jax.ShapeDtypeStruct((mp, n), jnp.float32),
    )(*args)


def _mm_lo(xa, xb, w1a, w1b, b1, w2, b2, tile=512):
    mp, kdim = xa.shape
    n1 = w1a.shape[1]
    n2 = w2.shape[1]
    grid = (mp // tile,)

    def body(a_ref, p_ref, w1a_ref, w1b_ref, b1_ref, w2_ref, b2_ref, o_ref):
        h = (jnp.dot(a_ref[...], w1a_ref[...],
                     preferred_element_type=jnp.float32)
             + jnp.dot(p_ref[...], w1b_ref[...],
                       preferred_element_type=jnp.float32) + b1_ref[...])
        h = jnp.where(h > 0, h, 0.1 * h)
        o_ref[...] = jnp.dot(h, w2_ref[...],
                             preferred_element_type=jnp.float32) + b2_ref[...]

    return pl.pallas_call(
        body,
        grid=grid,
        in_specs=[
            pl.BlockSpec((tile, kdim), lambda i: (i, 0)),
            pl.BlockSpec((tile, kdim), lambda i: (i, 0)),
            pl.BlockSpec((kdim, n1), lambda i: (0, 0)),
            pl.BlockSpec((kdim, n1), lambda i: (0, 0)),
            pl.BlockSpec((1, n1), lambda i: (0, 0)),
            pl.BlockSpec((n1, n2), lambda i: (0, 0)),
            pl.BlockSpec((1, n2), lambda i: (0, 0)),
        ],
        out_specs=pl.BlockSpec((tile, n2), lambda i: (i, 0)),
        out_shape=jax.ShapeDtypeStruct((mp, n2), jnp.float32),
    )(xa, xb, w1a, w1b, b1.reshape(1, n1), w2, b2.reshape(1, n2))


def _leaky(x):
    return jnp.where(x > 0, x, 0.1 * x)


def _bn_pad(x, m2, mask, leaky=False):
    xs = x[:m2]
    m = jnp.mean(xs, axis=0, keepdims=True)
    v = jnp.mean((xs - m) ** 2, axis=0, keepdims=True)
    y = (x - m) / jnp.sqrt(v + 1e-5)
    if leaky:
        y = _leaky(y)
    return y * mask


def _pad_rows(x, mp):
    return jnp.pad(x, ((0, mp - x.shape[0]), (0, 0)))


def _pad_idx(idx, n_pad):
    return jnp.pad(idx.astype(jnp.int32), (0, n_pad - idx.shape[0]))


def kernel(vox_feat, Win1, Wa1, Wb1, Win2, Wa2, Wb2, Wli, bli,
           Wpp1, bpp1, Wpp2, bpp2, Wpp3, bpp3, Wlo1, blo1, Wlo2, blo2,
           nbr, coors_inv_last, coors_inv, inv_ds):
    m2, c = vox_feat.shape
    n_pts = coors_inv.shape[0]
    m4 = min(32 ** 3, coors_inv.shape[0])
    mds = min(32 ** 3, inv_ds.shape[0])

    mp = ((m2 + 4095) // 4096) * 4096
    np_pts = ((n_pts + 4095) // 4096) * 4096
    mask = (jnp.arange(mp) < m2).astype(jnp.float32)[:, None]

    nbr_flat = jnp.pad(nbr.astype(jnp.int32),
                       ((0, mp - m2), (0, 0))).reshape(mp * 27 // _CH, _CH)
    cil_pad = _pad_idx(coors_inv_last, np_pts).reshape(np_pts // _CH, _CH)
    ci_pad = _pad_idx(coors_inv, np_pts).reshape(np_pts // _CH, _CH)

    wa1c = Wa1.reshape(27 * c, c)
    wb1c = Wb1.reshape(27 * c, c)
    wa2c = Wa2.reshape(27 * c, c)
    wb2c = Wb2.reshape(27 * c, c)

    def subm(x, wcat):
        g = _sc_gather_rows(x, nbr_flat)
        return _mm(g.reshape(mp, 27 * c), wcat)

    def block(x, win, wacat, wbcat):
        identity = _bn_pad(_mm(x, win), m2, mask)
        h = _bn_pad(subm(x, wacat), m2, mask, leaky=True)
        h = _bn_pad(subm(h, wbcat), m2, mask)
        return _leaky(h + identity) * mask

    x0 = _pad_rows(vox_feat, mp)
    v1 = block(x0, Win1, wa1c, wb1c)
    v_fea = block(v1, Win2, wa2c, wb2c)

    feats = x0 + v_fea
    ones = jnp.ones((m2, 1), jnp.float32)
    ds_cnt = jax.ops.segment_sum(ones, inv_ds, num_segments=mds)
    ds_sum = jax.ops.segment_sum(feats[:m2], inv_ds, num_segments=mds)
    ds = ds_sum / jnp.maximum(ds_cnt, 1.0)
    ds_mask = (ds_cnt > 0).astype(jnp.float32)

    identity = _mm(feats, Wli, bli, leaky=True)

    def bn_masked(x):
        n = jnp.sum(ds_mask)
        m = jnp.sum(x * ds_mask, axis=0, keepdims=True) / n
        v = jnp.sum(((x - m) ** 2) * ds_mask, axis=0, keepdims=True) / n
        return (x - m) / jnp.sqrt(v + 1e-5)

    pp = bn_masked(_mm(ds, Wpp1, bpp1, leaky=True, tile=512))
    pp = bn_masked(_mm(pp, Wpp2, bpp2, leaky=True, tile=512))
    pp = _mm(pp, Wpp3, bpp3, leaky=True, tile=512)

    comp_idx = jnp.take(inv_ds, coors_inv_last)
    comp_pad = _pad_idx(comp_idx, np_pts).reshape(np_pts // _CH, _CH)
    pts_a = _sc_gather_rows(identity, cil_pad)
    pts_b = _sc_gather_rows(pp, comp_pad)
    lo = _mm_lo(pts_a, pts_b, Wlo1[:c], Wlo1[c:], blo1, Wlo2, blo2)

    p_cnt = jax.ops.segment_sum(jnp.ones((n_pts, 1), jnp.float32),
                                coors_inv, num_segments=m4)
    p_sum = jax.ops.segment_sum(lo[:n_pts], coors_inv, num_segments=m4)
    p_fea = p_sum / jnp.maximum(p_cnt, 1.0)

    out = _sc_gather_rows(p_fea, ci_pad)
    return out[:n_pts]
